# baseline (speedup 1.0000x reference)
import functools

import jax
import jax.numpy as jnp
from jax import lax
from jax.experimental import pallas as pl
from jax.experimental.pallas import tpu as pltpu
from jax.experimental.pallas import tpu_sc as plsc

VOCAB = 1000
D = 64
BATCH = 1024
T = 50
BT = BATCH * T

NC = 2
NS = 16
NW = NC * NS
PER_W = BT // NW
CHUNK = 80
NCHUNK = PER_W // CHUNK


def _sc_gather(idx_flat, wte):
    mesh = plsc.VectorSubcoreMesh(core_axis_name="c", subcore_axis_name="s")

    @functools.partial(
        pl.kernel,
        mesh=mesh,
        out_type=jax.ShapeDtypeStruct((BT, D), jnp.float32),
        scratch_types=[
            pltpu.VMEM((PER_W,), jnp.int32),
            pltpu.VMEM((PER_W, D), jnp.float32),
            pltpu.SemaphoreType.DMA,
        ],
        compiler_params=pltpu.CompilerParams(use_tc_tiling_on_sc=False),
    )
    def k(idx_hbm, wte_hbm, out_hbm, idx_v, rows_v, sem):
        wid = lax.axis_index("s") * NC + lax.axis_index("c")
        base = wid * PER_W
        pltpu.sync_copy(idx_hbm.at[pl.ds(base, PER_W)], idx_v)
        copies = []
        for j in range(NCHUNK):
            copies.append(pltpu.async_copy(
                wte_hbm.at[idx_v.at[pl.ds(j * CHUNK, CHUNK)]],
                rows_v.at[pl.ds(j * CHUNK, CHUNK)],
                sem))
        for c in copies:
            c.wait()
        pltpu.sync_copy(rows_v, out_hbm.at[pl.ds(base, PER_W)])

    return k(idx_flat, wte)


ROWS = 800
GRID = BT // ROWS


def _mlp_body(e_ref, w1_ref, b1_ref, w2_ref, b2_ref, out_ref):
    e = e_ref[...]
    tmod = lax.broadcasted_iota(jnp.int32, (ROWS, D), 0) % T
    e1 = jnp.where(tmod >= 1, jnp.roll(e, 1, axis=0), 0.0)
    e2 = jnp.where(tmod >= 2, jnp.roll(e, 2, axis=0), 0.0)
    hidden = (e + e1 + e2) * (1.0 / 3.0)
    h = jnp.tanh(
        jnp.dot(hidden.astype(jnp.bfloat16), w1_ref[...],
                preferred_element_type=jnp.float32)
        + b1_ref[...])
    out_ref[...] = (
        jnp.dot(h.astype(jnp.bfloat16), w2_ref[...],
                preferred_element_type=jnp.float32)
        + b2_ref[...])


def _mlp(e, W1, b1, W2, b2):
    return pl.pallas_call(
        _mlp_body,
        grid=(GRID,),
        in_specs=[
            pl.BlockSpec((ROWS, D), lambda i: (i, 0)),
            pl.BlockSpec((D, D), lambda i: (0, 0)),
            pl.BlockSpec((1, D), lambda i: (0, 0)),
            pl.BlockSpec((D, VOCAB), lambda i: (0, 0)),
            pl.BlockSpec((1, VOCAB), lambda i: (0, 0)),
        ],
        out_specs=pl.BlockSpec((ROWS, VOCAB), lambda i: (i, 0)),
        out_shape=jax.ShapeDtypeStruct((BT, VOCAB), jnp.float32),
    )(e, W1.astype(jnp.bfloat16), b1.reshape(1, D),
      W2.astype(jnp.bfloat16), b2.reshape(1, VOCAB))


def kernel(idx, wte, W1, b1, W2, b2):
    wte_eff = wte.at[0].set(0.0)
    idx_flat = idx.reshape(BT)
    e = _sc_gather(idx_flat, wte_eff)
    out = _mlp(e, W1, b1, W2, b2)
    return out.reshape(BATCH, T, VOCAB)


# speedup vs baseline: 1.2549x; 1.2549x over previous
import functools

import jax
import jax.numpy as jnp
from jax import lax
from jax.experimental import pallas as pl
from jax.experimental.pallas import tpu as pltpu
from jax.experimental.pallas import tpu_sc as plsc

VOCAB = 1000
D = 64
BATCH = 1024
T = 50
BT = BATCH * T

NC = 2
NS = 16
NW = NC * NS
PER_W = BT // NW
CHUNK = 80
NCHUNK = PER_W // CHUNK


def _sc_gather(idx_flat, wte):
    mesh = plsc.VectorSubcoreMesh(core_axis_name="c", subcore_axis_name="s")

    @functools.partial(
        pl.kernel,
        mesh=mesh,
        out_type=jax.ShapeDtypeStruct((BT, D), jnp.float32),
        scratch_types=[
            pltpu.VMEM((PER_W,), jnp.int32),
            pltpu.VMEM((PER_W, D), jnp.float32),
            pltpu.SemaphoreType.DMA,
        ],
        compiler_params=pltpu.CompilerParams(use_tc_tiling_on_sc=False),
    )
    def k(idx_hbm, wte_hbm, out_hbm, idx_v, rows_v, sem):
        wid = lax.axis_index("s") * NC + lax.axis_index("c")
        base = wid * PER_W
        pltpu.sync_copy(idx_hbm.at[pl.ds(base, PER_W)], idx_v)
        copies = []
        for j in range(NCHUNK):
            copies.append(pltpu.async_copy(
                wte_hbm.at[idx_v.at[pl.ds(j * CHUNK, CHUNK)]],
                rows_v.at[pl.ds(j * CHUNK, CHUNK)],
                sem))
        for c in copies:
            c.wait()
        pltpu.sync_copy(rows_v, out_hbm.at[pl.ds(base, PER_W)])

    return k(idx_flat, wte)


BB = 16
ROWS = BB * T
GRID = BATCH // BB


def _mlp_body(e_ref, w1_ref, b1_ref, w2_ref, b2_ref, out_ref):
    e = e_ref[...]
    tmod = lax.broadcasted_iota(jnp.int32, (ROWS, D), 0) % T
    e1 = jnp.where(tmod >= 1, jnp.roll(e, 1, axis=0), 0.0)
    e2 = jnp.where(tmod >= 2, jnp.roll(e, 2, axis=0), 0.0)
    hidden = (e + e1 + e2) * (1.0 / 3.0)
    h = jnp.tanh(
        jnp.dot(hidden.astype(jnp.bfloat16), w1_ref[...],
                preferred_element_type=jnp.float32)
        + b1_ref[...])
    logits = (
        jnp.dot(h.astype(jnp.bfloat16), w2_ref[...],
                preferred_element_type=jnp.float32)
        + b2_ref[...])
    for j in range(BB):
        out_ref[j] = logits[j * T:(j + 1) * T, :]


def _mlp(e, W1, b1, W2, b2):
    return pl.pallas_call(
        _mlp_body,
        grid=(GRID,),
        in_specs=[
            pl.BlockSpec((ROWS, D), lambda i: (i, 0)),
            pl.BlockSpec((D, D), lambda i: (0, 0)),
            pl.BlockSpec((1, D), lambda i: (0, 0)),
            pl.BlockSpec((D, VOCAB), lambda i: (0, 0)),
            pl.BlockSpec((1, VOCAB), lambda i: (0, 0)),
        ],
        out_specs=pl.BlockSpec((BB, T, VOCAB), lambda i: (i, 0, 0)),
        out_shape=jax.ShapeDtypeStruct((BATCH, T, VOCAB), jnp.float32),
    )(e, W1.astype(jnp.bfloat16), b1.reshape(1, D),
      W2.astype(jnp.bfloat16), b2.reshape(1, VOCAB))


def kernel(idx, wte, W1, b1, W2, b2):
    wte_eff = wte.at[0].set(0.0)
    idx_flat = idx.reshape(BT)
    e = _sc_gather(idx_flat, wte_eff)
    return _mlp(e, W1, b1, W2, b2)
